# baseline (speedup 1.0000x reference)
import functools

import jax
import jax.numpy as jnp
from jax import lax
from jax.experimental import pallas as pl
from jax.experimental.pallas import tpu as pltpu
from jax.experimental.pallas import tpu_sc as plsc

_CLASSES = 1000
_ROWS = 1024 * 26
_NC, _NS = 2, 16
_NW = _NC * _NS
_RPW = _ROWS // _NW
_CHUNK = 16
_NCHUNK = _RPW // _CHUNK
_CELEMS = _CHUNK * _CLASSES

_mesh = plsc.VectorSubcoreMesh(core_axis_name="c", subcore_axis_name="s")


@functools.partial(
    pl.kernel,
    mesh=_mesh,
    out_type=jax.ShapeDtypeStruct((_ROWS * _CLASSES,), jnp.float32),
    scratch_types=[
        pltpu.VMEM((_RPW,), jnp.int32),
        pltpu.VMEM((_CELEMS,), jnp.float32),
    ],
    compiler_params=pltpu.CompilerParams(needs_layout_passes=False),
)
def _onehot_sc(idx_hbm, noise_hbm, out_hbm, idx_v, buf):
    wid = lax.axis_index("s") * _NC + lax.axis_index("c")
    base = wid * _RPW
    pltpu.sync_copy(idx_hbm.at[pl.ds(base, _RPW)], idx_v)

    lanes = lax.iota(jnp.int32, 16) * _CLASSES
    ones = jnp.full((16,), 1.0, jnp.float32)

    def chunk_body(g, carry):
        off = (base + g * _CHUNK) * _CLASSES
        pltpu.sync_copy(noise_hbm.at[pl.ds(off, _CELEMS)], buf)

        def scale_body(j, c):
            s0 = j * 128
            for u in range(8):
                s = s0 + u * 16
                buf[pl.ds(s, 16)] = buf[pl.ds(s, 16)] * 0.01
            return c

        lax.fori_loop(0, _CELEMS // 128, scale_body, 0)
        offs = lanes + idx_v[pl.ds(g * _CHUNK, _CHUNK)]
        plsc.addupdate_scatter(buf, [offs], ones)
        pltpu.sync_copy(buf, out_hbm.at[pl.ds(off, _CELEMS)])
        return carry

    lax.fori_loop(0, _NCHUNK, chunk_body, 0)


def kernel(indices, noise):
    idx = indices.reshape(-1).astype(jnp.int32)
    out = _onehot_sc(idx, noise.reshape(-1))
    return out.reshape(noise.shape)


# speedup vs baseline: 1.1009x; 1.1009x over previous
import functools

import jax
import jax.numpy as jnp
from jax import lax
from jax.experimental import pallas as pl
from jax.experimental.pallas import tpu as pltpu
from jax.experimental.pallas import tpu_sc as plsc

_CLASSES = 1000
_ROWS = 1024 * 26
_NC, _NS = 2, 16
_NW = _NC * _NS
_RPW = _ROWS // _NW
_CHUNK = 16
_NCHUNK = _RPW // _CHUNK
_CELEMS = _CHUNK * _CLASSES

_mesh = plsc.VectorSubcoreMesh(core_axis_name="c", subcore_axis_name="s")


@functools.partial(
    pl.kernel,
    mesh=_mesh,
    out_type=jax.ShapeDtypeStruct((_ROWS * _CLASSES,), jnp.float32),
    scratch_types=[
        pltpu.VMEM((_RPW,), jnp.int32),
        pltpu.VMEM((_CELEMS,), jnp.float32),
        pltpu.VMEM((_CELEMS,), jnp.float32),
        pltpu.VMEM((_CELEMS,), jnp.float32),
        pltpu.VMEM((_CELEMS,), jnp.float32),
        pltpu.SemaphoreType.DMA,
        pltpu.SemaphoreType.DMA,
        pltpu.SemaphoreType.DMA,
        pltpu.SemaphoreType.DMA,
    ],
    compiler_params=pltpu.CompilerParams(needs_layout_passes=False),
)
def _onehot_sc(idx_hbm, noise_hbm, out_hbm, idx_v, inb0, inb1, outb0, outb1,
               isem0, isem1, osem0, osem1):
    wid = lax.axis_index("s") * _NC + lax.axis_index("c")
    base = wid * _RPW
    pltpu.sync_copy(idx_hbm.at[pl.ds(base, _RPW)], idx_v)

    lanes = lax.iota(jnp.int32, 16) * _CLASSES
    ones = jnp.full((16,), 1.0, jnp.float32)

    pltpu.make_async_copy(
        noise_hbm.at[pl.ds(base * _CLASSES, _CELEMS)], inb0, isem0).start()
    pltpu.make_async_copy(
        noise_hbm.at[pl.ds((base + _CHUNK) * _CLASSES, _CELEMS)], inb1,
        isem1).start()

    def chunk_pair(gg, carry):
        for b, (ib, ob, isem, osem) in enumerate((
                (inb0, outb0, isem0, osem0), (inb1, outb1, isem1, osem1))):
            g = gg * 2 + b
            off = (base + g * _CHUNK) * _CLASSES

            @pl.when(gg >= 1)
            def _wait_out():
                pltpu.make_async_copy(
                    ob, out_hbm.at[pl.ds(0, _CELEMS)], osem).wait()

            pltpu.make_async_copy(
                noise_hbm.at[pl.ds(0, _CELEMS)], ib, isem).wait()

            @plsc.parallel_loop(0, _CELEMS, step=16, unroll=8)
            def _scale(i):
                ob[pl.ds(i, 16)] = ib[pl.ds(i, 16)] * 0.01

            @pl.when(gg <= _NCHUNK // 2 - 2)
            def _prefetch():
                off2 = (base + (g + 2) * _CHUNK) * _CLASSES
                pltpu.make_async_copy(
                    noise_hbm.at[pl.ds(off2, _CELEMS)], ib, isem).start()

            offs = lanes + idx_v[pl.ds(g * _CHUNK, _CHUNK)]
            plsc.addupdate_scatter(ob, [offs], ones)
            pltpu.make_async_copy(
                ob, out_hbm.at[pl.ds(off, _CELEMS)], osem).start()
        return carry

    lax.fori_loop(0, _NCHUNK // 2, chunk_pair, 0)
    pltpu.make_async_copy(outb0, out_hbm.at[pl.ds(0, _CELEMS)], osem0).wait()
    pltpu.make_async_copy(outb1, out_hbm.at[pl.ds(0, _CELEMS)], osem1).wait()


def kernel(indices, noise):
    idx = indices.reshape(-1).astype(jnp.int32)
    out = _onehot_sc(idx, noise.reshape(-1))
    return out.reshape(noise.shape)


# speedup vs baseline: 1.9646x; 1.7846x over previous
import functools

import jax
import jax.numpy as jnp
from jax import lax
from jax.experimental import pallas as pl
from jax.experimental.pallas import tpu as pltpu
from jax.experimental.pallas import tpu_sc as plsc

_B = 1024
_M = 26
_CLASSES = 1000
_NC, _NS = 2, 16
_NW = _NC * _NS
_BPW = _B // _NW
_NBUF = 3

_mesh = plsc.VectorSubcoreMesh(core_axis_name="c", subcore_axis_name="s")


def _scale_block(buf):
    lanesel = lax.iota(jnp.int32, 16) < 8

    @plsc.parallel_loop(0, 62, unroll=2)
    def _cols(c):
        s = c * 16
        for r in range(_M):
            buf[r, pl.ds(s, 16)] = buf[r, pl.ds(s, 16)] * 0.01

    for r in range(_M):
        v = buf[r, pl.ds(984, 16)]
        buf[r, pl.ds(984, 16)] = jnp.where(lanesel, v, v * 0.01)


@functools.partial(
    pl.kernel,
    mesh=_mesh,
    out_type=jax.ShapeDtypeStruct((_B, _M, _CLASSES), jnp.float32),
    scratch_types=[
        pltpu.VMEM((_BPW, 32), jnp.int32),
        pltpu.VMEM((_M, _CLASSES), jnp.float32),
        pltpu.VMEM((_M, _CLASSES), jnp.float32),
        pltpu.VMEM((_M, _CLASSES), jnp.float32),
        pltpu.SemaphoreType.DMA,
        pltpu.SemaphoreType.DMA,
        pltpu.SemaphoreType.DMA,
        pltpu.SemaphoreType.DMA,
        pltpu.SemaphoreType.DMA,
        pltpu.SemaphoreType.DMA,
    ],
    compiler_params=pltpu.CompilerParams(needs_layout_passes=False),
)
def _onehot_sc(idx_hbm, noise_hbm, out_hbm, idx_v, buf0, buf1, buf2,
               is0, is1, is2, os0, os1, os2):
    wid = lax.axis_index("s") * _NC + lax.axis_index("c")
    base = wid * _BPW
    pltpu.sync_copy(idx_hbm.at[pl.ds(base, _BPW), :], idx_v)

    bufs = (buf0, buf1, buf2)
    isems = (is0, is1, is2)
    osems = (os0, os1, os2)

    r0 = lax.iota(jnp.int32, 16)
    r1 = r0 + 16
    rowmask = r1 < _M
    ones = jnp.full((16,), 1.0, jnp.float32)

    for k in range(2):
        pltpu.make_async_copy(noise_hbm.at[base + k], bufs[k], isems[k]).start()

    def group(gg, carry):
        for b3 in range(_NBUF):
            k = gg * _NBUF + b3
            buf, isem, osem = bufs[b3], isems[b3], osems[b3]
            bufd, isemd, osemd = (bufs[(b3 + 2) % 3], isems[(b3 + 2) % 3],
                                  osems[(b3 + 2) % 3])

            @pl.when(k < _BPW)
            def _compute():
                pltpu.make_async_copy(noise_hbm.at[0], buf, isem).wait()
                _scale_block(buf)

            @pl.when((k >= 1) & (k < _BPW + 1))
            def _retire():
                pltpu.make_async_copy(bufd, out_hbm.at[0], osemd).wait()

            @pl.when(k + 2 < _BPW)
            def _prefetch():
                pltpu.make_async_copy(
                    noise_hbm.at[base + k + 2], bufd, isemd).start()

            @pl.when(k < _BPW)
            def _scatter_store():
                c0 = idx_v[k, pl.ds(0, 16)]
                plsc.addupdate_scatter(buf, [r0, c0], ones)
                c1 = idx_v[k, pl.ds(16, 16)]
                plsc.addupdate_scatter(buf, [r1, c1], ones, mask=rowmask)
                pltpu.make_async_copy(buf, out_hbm.at[base + k], osem).start()
        return carry

    lax.fori_loop(0, (_BPW + _NBUF - 1) // _NBUF + 1, group, 0)


def kernel(indices, noise):
    idx = jnp.pad(indices.astype(jnp.int32), ((0, 0), (0, 32 - _M)))
    out = _onehot_sc(idx, noise)
    return out


# speedup vs baseline: 6.2420x; 3.1772x over previous
import functools

import jax
import jax.numpy as jnp
from jax import lax
from jax.experimental import pallas as pl
from jax.experimental.pallas import tpu as pltpu
from jax.experimental.pallas import tpu_sc as plsc

_B = 1024
_M = 26
_CLASSES = 1000
_NC, _NS = 2, 16
_NW = _NC * _NS
_TROWS = _M * (_CLASSES // 8)
_PER_W = _TROWS // _NW
_EXTRA = _TROWS - _PER_W * _NW
_JPM = _CLASSES // 8

_mesh = plsc.VectorSubcoreMesh(core_axis_name="c", subcore_axis_name="s")


@functools.partial(
    pl.kernel,
    mesh=_mesh,
    out_type=jax.ShapeDtypeStruct((_M, _CLASSES, _B), jnp.float32),
    scratch_types=[
        pltpu.VMEM((2, _B), jnp.int32),
        pltpu.VMEM((8, _B), jnp.float32),
        pltpu.VMEM((8, _B), jnp.float32),
        pltpu.VMEM((8, _B), jnp.float32),
        pltpu.SemaphoreType.DMA,
        pltpu.SemaphoreType.DMA,
        pltpu.SemaphoreType.DMA,
        pltpu.SemaphoreType.DMA,
        pltpu.SemaphoreType.DMA,
        pltpu.SemaphoreType.DMA,
    ],
    compiler_params=pltpu.CompilerParams(needs_layout_passes=False),
)
def _onehot_sc(idx_hbm, noise_hbm, out_hbm, idx_v, buf0, buf1, buf2,
               is0, is1, is2, os0, os1, os2):
    wid = lax.axis_index("s") * _NC + lax.axis_index("c")
    base = wid * _PER_W + jnp.minimum(wid, _EXTRA)
    cnt = _PER_W + jnp.where(wid < _EXTRA, 1, 0)

    m_lo = base // _JPM
    m_hi = jnp.minimum(m_lo + 1, _M - 1)
    pltpu.sync_copy(idx_hbm.at[m_lo, :], idx_v.at[0, :])
    pltpu.sync_copy(idx_hbm.at[m_hi, :], idx_v.at[1, :])

    bufs = (buf0, buf1, buf2)
    isems = (is0, is1, is2)
    osems = (os0, os1, os2)

    def _src(k):
        t = base + k
        m = t // _JPM
        j = t - m * _JPM
        return noise_hbm.at[m, pl.ds(j * 8, 8), :], m, j

    def _dst(k):
        t = base + k
        m = t // _JPM
        j = t - m * _JPM
        return out_hbm.at[m, pl.ds(j * 8, 8), :]

    for k in range(2):
        src, _, _ = _src(k)
        pltpu.make_async_copy(src, bufs[k], isems[k]).start()

    def group(gg, carry):
        for b3 in range(3):
            k = gg * 3 + b3
            buf, isem, osem = bufs[b3], isems[b3], osems[b3]
            bufd, isemd, osemd = (bufs[(b3 + 2) % 3], isems[(b3 + 2) % 3],
                                  osems[(b3 + 2) % 3])

            @pl.when(k < cnt)
            def _compute():
                pltpu.make_async_copy(noise_hbm.at[0, pl.ds(0, 8), :], buf,
                                      isem).wait()
                t = base + k
                m = t // _JPM
                c_base = (t - m * _JPM) * 8
                r = m - m_lo

                @plsc.parallel_loop(0, _B // 16, unroll=2)
                def _blk(blk):
                    b0 = blk * 16
                    idxv = idx_v[r, pl.ds(b0, 16)]
                    for row in range(8):
                        v = buf[row, pl.ds(b0, 16)]
                        hot = jnp.where(idxv == c_base + row, 1.0, 0.0)
                        buf[row, pl.ds(b0, 16)] = v * 0.01 + hot

            @pl.when((k >= 1) & (k < cnt + 1))
            def _retire():
                pltpu.make_async_copy(bufd, out_hbm.at[0, pl.ds(0, 8), :],
                                      osemd).wait()

            @pl.when(k + 2 < cnt)
            def _prefetch():
                src, _, _ = _src(k + 2)
                pltpu.make_async_copy(src, bufd, isemd).start()

            @pl.when(k < cnt)
            def _store():
                pltpu.make_async_copy(buf, _dst(k), osem).start()
        return carry

    lax.fori_loop(0, (_PER_W + 1 + 2) // 3 + 1, group, 0)


def kernel(indices, noise):
    idx_t = jnp.transpose(indices.astype(jnp.int32))
    noise_t = jnp.transpose(noise, (1, 2, 0))
    out_t = _onehot_sc(idx_t, noise_t)
    return jnp.transpose(out_t, (2, 0, 1))
